# Initial kernel scaffold; baseline (speedup 1.0000x reference)
import jax
import jax.numpy as jnp
from jax.experimental import pallas as pl


def kernel(id, W):
    raise NotImplementedError("write your pallas kernel here")


# speedup vs baseline: 4.0914x; 4.0914x over previous
import functools

import jax
import jax.numpy as jnp
from jax import lax
from jax.experimental import pallas as pl
from jax.experimental.pallas import tpu as pltpu
from jax.experimental.pallas import tpu_sc as plsc

NUM_CORES = 2
NUM_SUBCORES = 16
NW = NUM_CORES * NUM_SUBCORES
CHUNK = 128


@functools.partial(jax.jit, static_argnames=())
def _embed(idx3, W):
    n_chunks = idx3.shape[1]
    b_per_w = n_chunks * CHUNK
    total = NW * b_per_w
    D = W.shape[1]
    mesh = plsc.VectorSubcoreMesh(
        core_axis_name="c", subcore_axis_name="s",
        num_cores=NUM_CORES, num_subcores=NUM_SUBCORES)

    @functools.partial(
        pl.kernel,
        mesh=mesh,
        out_type=jax.ShapeDtypeStruct((total, D), jnp.float32),
        scratch_types=[
            pltpu.VMEM((n_chunks, CHUNK), jnp.int32),
            pltpu.VMEM((CHUNK, D), jnp.float32),
            pltpu.SemaphoreType.DMA,
        ],
        compiler_params=pltpu.CompilerParams(use_tc_tiling_on_sc=False),
    )
    def k(table_hbm, idx_hbm, out_hbm, idx_v, rows_v, sem):
        wid = lax.axis_index("s") * NUM_CORES + lax.axis_index("c")
        base = wid * b_per_w
        pltpu.sync_copy(idx_hbm.at[wid], idx_v)

        def body(j, carry):
            pltpu.async_copy(table_hbm.at[idx_v.at[j]], rows_v, sem).wait()
            pltpu.sync_copy(rows_v, out_hbm.at[pl.ds(base + j * CHUNK, CHUNK)])
            return carry

        lax.fori_loop(0, n_chunks, body, 0)

    return k(W, idx3)


def kernel(id, W):
    B, S = id.shape
    D = W.shape[1]
    total = B * S
    idx3 = id.reshape(NW, total // (NW * CHUNK), CHUNK).astype(jnp.int32)
    out = _embed(idx3, W)
    return out.reshape(B, S, D)


# speedup vs baseline: 4.6750x; 1.1426x over previous
import functools

import jax
import jax.numpy as jnp
from jax import lax
from jax.experimental import pallas as pl
from jax.experimental.pallas import tpu as pltpu
from jax.experimental.pallas import tpu_sc as plsc

NUM_CORES = 2
NUM_SUBCORES = 16
NW = NUM_CORES * NUM_SUBCORES
CHUNK = 128
NBUF = 5


@functools.partial(jax.jit, static_argnames=())
def _embed(idx3, W):
    n_chunks = idx3.shape[1]
    b_per_w = n_chunks * CHUNK
    total = NW * b_per_w
    D = W.shape[1]
    n_outer = n_chunks // NBUF
    assert n_chunks % NBUF == 0 and n_outer >= 2
    mesh = plsc.VectorSubcoreMesh(
        core_axis_name="c", subcore_axis_name="s",
        num_cores=NUM_CORES, num_subcores=NUM_SUBCORES)

    @functools.partial(
        pl.kernel,
        mesh=mesh,
        out_type=jax.ShapeDtypeStruct((total, D), jnp.float32),
        scratch_types=[
            pltpu.VMEM((n_chunks, CHUNK), jnp.int32),
            pltpu.VMEM((NBUF, CHUNK, D), jnp.float32),
        ] + [pltpu.SemaphoreType.DMA] * (2 * NBUF),
        compiler_params=pltpu.CompilerParams(use_tc_tiling_on_sc=False),
    )
    def k(table_hbm, idx_hbm, out_hbm, idx_v, bufs, *sems):
        gsem = sems[:NBUF]
        ssem = sems[NBUF:]
        wid = lax.axis_index("s") * NUM_CORES + lax.axis_index("c")
        base = wid * b_per_w
        pltpu.sync_copy(idx_hbm.at[wid], idx_v)

        def gather(j, b):
            pltpu.async_copy(table_hbm.at[idx_v.at[j]], bufs.at[b], gsem[b])

        def store(j, b):
            pltpu.async_copy(
                bufs.at[b], out_hbm.at[pl.ds(base + j * CHUNK, CHUNK)],
                ssem[b])

        def wait_gather(j, b):
            pltpu.make_async_copy(
                table_hbm.at[idx_v.at[j]], bufs.at[b], gsem[b]).wait()

        def wait_store(j, b):
            pltpu.make_async_copy(
                bufs.at[b], out_hbm.at[pl.ds(base + j * CHUNK, CHUNK)],
                ssem[b]).wait()

        for b in range(NBUF):
            gather(b, b)

        def body(g, carry):
            for b in range(NBUF):
                j = g * NBUF + b
                wait_gather(j, b)
                store(j, b)
                wait_store(j, b)
                gather(j + NBUF, b)
            return carry

        lax.fori_loop(0, n_outer - 1, body, 0)

        for b in range(NBUF):
            j = (n_outer - 1) * NBUF + b
            wait_gather(j, b)
            store(j, b)
        for b in range(NBUF):
            j = (n_outer - 1) * NBUF + b
            wait_store(j, b)

    return k(W, idx3)


def kernel(id, W):
    B, S = id.shape
    D = W.shape[1]
    total = B * S
    idx3 = id.reshape(NW, total // (NW * CHUNK), CHUNK).astype(jnp.int32)
    out = _embed(idx3, W)
    return out.reshape(B, S, D)
